# baseline (speedup 1.0000x reference)
import jax
import jax.numpy as jnp
from jax import lax
from jax.experimental import pallas as pl
from jax.experimental.pallas import tpu as pltpu
from jax.experimental.pallas import tpu_sc as plsc

N = 100000
C = 128
G = 64
ROW = 80
NCORE = 2
NPC = N // NCORE
NSLOTS = NPC // ROW
NSUB = 16
GPT = G // NSUB
NBUF = 6
LOOK = 3
TOTJ = -(-NSLOTS // NSUB)
STEPS = -(-TOTJ // NBUF)


def _body(x_hbm, b_hbm, ps_hbm, ct_hbm, idx_v, x_v, cnt_v, iota_v, zero_v,
          sums4_v, sums_sh, cnt_sh, dsem, ssem):
    cid = lax.axis_index("c")
    sid = lax.axis_index("s")
    r0 = cid * NPC
    g0 = sid * GPT

    zeros16f = jnp.zeros((16,), jnp.float32)
    ones16f = jnp.ones((16,), jnp.float32)

    for g in range(GPT):
        for j in range(C // 16):
            zero_v[g, pl.ds(j * 16, 16)] = zeros16f
    for j in range(G // 16):
        cnt_v[pl.ds(j * 16, 16)] = zeros16f
        iota_v[pl.ds(j * 16, 16)] = lax.iota(jnp.int32, 16) + (16 * j)
    pltpu.sync_copy(zero_v, sums_sh.at[pl.ds(g0, GPT)])

    @pl.when(sid == 0)
    def _():
        pltpu.sync_copy(zero_v.at[0, pl.ds(0, G)], cnt_sh)

    plsc.subcore_barrier()

    def issue(j, b):
        slot = j * NSUB + sid

        @pl.when(slot < NSLOTS)
        def _():
            off = r0 + slot * ROW
            pltpu.async_copy(b_hbm.at[pl.ds(off, ROW)], idx_v[b], dsem[b])
            pltpu.async_copy(x_hbm.at[pl.ds(off, ROW), :], x_v[b], dsem[b])

    def process(j, b):
        slot = j * NSUB + sid

        @pl.when(slot < NSLOTS)
        def _():
            off = r0 + slot * ROW
            pltpu.make_async_copy(
                b_hbm.at[pl.ds(off, ROW)], idx_v[b], dsem[b]).wait()
            pltpu.make_async_copy(
                x_hbm.at[pl.ds(off, ROW), :], x_v[b], dsem[b]).wait()
            pltpu.async_copy(x_v[b], sums_sh.at[idx_v[b]], ssem[b], add=True)
            for q in range(ROW // 16):
                iv = idx_v[b][pl.ds(q * 16, 16)]
                plsc.addupdate_scatter(cnt_v, [iv], ones16f)

    def drain(j, b):
        slot = j * NSUB + sid

        @pl.when((j >= 0) & (slot < NSLOTS))
        def _():
            pltpu.make_async_copy(
                x_v[b], sums_sh.at[idx_v[b]], ssem[b]).wait()

    for j in range(LOOK):
        issue(j, j % NBUF)

    def step(i, _):
        for b in range(NBUF):
            j = i * NBUF + b
            process(j, b)
            drain(j - (NBUF - LOOK), (b + LOOK) % NBUF)
            issue(j + LOOK, (b + LOOK) % NBUF)
        return _

    lax.fori_loop(0, STEPS, step, None)
    for t in range(NBUF - LOOK):
        jj = STEPS * NBUF - (NBUF - LOOK) + t
        drain(jj, jj % NBUF)

    pltpu.sync_copy(cnt_v, cnt_sh.at[iota_v], add=True)
    plsc.subcore_barrier()

    pltpu.sync_copy(sums_sh.at[pl.ds(g0, GPT)], sums4_v)
    pltpu.sync_copy(sums4_v, ps_hbm.at[cid, pl.ds(g0, GPT), :])

    @pl.when(sid == 0)
    def _():
        pltpu.sync_copy(cnt_sh, cnt_v)
        pltpu.sync_copy(cnt_v, ct_hbm.at[cid])


def _tc_body(ps_ref, ct_ref, o_ref):
    s = ps_ref[0] + ps_ref[1]
    c = ct_ref[0, 0] + ct_ref[0, 1]
    o_ref[...] = s / c[:, None]


@jax.jit
def _pooling(x, batch):
    mesh = plsc.VectorSubcoreMesh(core_axis_name="c", subcore_axis_name="s")
    f = pl.kernel(
        _body,
        out_type=[
            jax.ShapeDtypeStruct((NCORE, G, C), jnp.float32),
            jax.ShapeDtypeStruct((NCORE, G), jnp.float32),
        ],
        mesh=mesh,
        compiler_params=pltpu.CompilerParams(use_tc_tiling_on_sc=False,
                                             needs_layout_passes=False,
                                             disable_bounds_checks=True,
                                             disable_semaphore_checks=True),
        scratch_types=[
            [pltpu.VMEM((ROW,), jnp.int32) for _ in range(NBUF)],
            [pltpu.VMEM((ROW, C), jnp.float32) for _ in range(NBUF)],
            pltpu.VMEM((G,), jnp.float32),
            pltpu.VMEM((G,), jnp.int32),
            pltpu.VMEM((GPT, C), jnp.float32),
            pltpu.VMEM((GPT, C), jnp.float32),
            pltpu.VMEM_SHARED((G, C), jnp.float32),
            pltpu.VMEM_SHARED((G,), jnp.float32),
            [pltpu.SemaphoreType.DMA for _ in range(NBUF)],
            [pltpu.SemaphoreType.DMA for _ in range(NBUF)],
        ],
    )
    psums, cnts = f(x, batch)
    combine = pl.pallas_call(
        _tc_body,
        out_shape=jax.ShapeDtypeStruct((G, C), jnp.float32),
    )
    return combine(psums, cnts.reshape(1, NCORE, G))


def kernel(x, batch):
    return _pooling(x, batch.astype(jnp.int32))


# speedup vs baseline: 1.2034x; 1.2034x over previous
import jax
import jax.numpy as jnp
from jax import lax
from jax.experimental import pallas as pl
from jax.experimental.pallas import tpu as pltpu
from jax.experimental.pallas import tpu_sc as plsc

N = 100000
C = 128
G = 64
ROW = 80
SUBC = 5
CHUNK = ROW * SUBC
NSLOTS = N // CHUNK
NSUB = 16
NCORE = 2
CH = C // NCORE
GPT = G // NSUB
NBUF = 4
TOTJ = -(-NSLOTS // NSUB)


def _body(x_hbm, b_hbm, out_hbm, idx_v, x_v, cnt_v, iota_v, zero_v,
          sums4_v, cnt64_v, out_v, sums_sh, cnt_sh, dsem, ssem):
    cid = lax.axis_index("c")
    sid = lax.axis_index("s")
    c0 = cid * CH
    g0 = sid * GPT

    zeros16f = jnp.zeros((16,), jnp.float32)
    ones16f = jnp.ones((16,), jnp.float32)

    for g in range(GPT):
        for j in range(CH // 16):
            zero_v[g, pl.ds(j * 16, 16)] = zeros16f
    for j in range(G // 16):
        cnt_v[pl.ds(j * 16, 16)] = zeros16f
        iota_v[pl.ds(j * 16, 16)] = lax.iota(jnp.int32, 16) + (16 * j)
    pltpu.sync_copy(zero_v, sums_sh.at[pl.ds(g0, GPT)])

    @pl.when(sid == 0)
    def _():
        pltpu.sync_copy(zero_v.at[0], cnt_sh)

    plsc.subcore_barrier()

    def issue(j, b):
        slot = j * NSUB + sid

        @pl.when(slot < NSLOTS)
        def _():
            pltpu.async_copy(
                b_hbm.at[pl.ds(slot * SUBC, SUBC), :], idx_v[b], dsem[b])
            pltpu.async_copy(
                x_hbm.at[pl.ds(slot * CHUNK, CHUNK), pl.ds(c0, CH)],
                x_v[b], dsem[b])

    def process(j, b):
        slot = j * NSUB + sid

        @pl.when(slot < NSLOTS)
        def _():
            pltpu.make_async_copy(
                b_hbm.at[pl.ds(slot * SUBC, SUBC), :], idx_v[b],
                dsem[b]).wait()
            pltpu.make_async_copy(
                x_hbm.at[pl.ds(slot * CHUNK, CHUNK), pl.ds(c0, CH)], x_v[b],
                dsem[b]).wait()
            for k in range(SUBC):
                pltpu.async_copy(
                    x_v[b].at[pl.ds(k * ROW, ROW)],
                    sums_sh.at[idx_v[b].at[k]], ssem[b], add=True)
            for k in range(SUBC):
                for q in range(ROW // 16):
                    iv = idx_v[b][k, pl.ds(q * 16, 16)]
                    plsc.addupdate_scatter(cnt_v, [iv], ones16f)

    def drain(j, b):
        slot = j * NSUB + sid

        @pl.when((j >= 0) & (slot < NSLOTS))
        def _():
            for k in range(SUBC):
                pltpu.make_async_copy(
                    x_v[b].at[pl.ds(k * ROW, ROW)],
                    sums_sh.at[idx_v[b].at[k]], ssem[b]).wait()

    LOOK = 3

    for j in range(LOOK):
        issue(j, j % NBUF)

    def step(i, _):
        for b in range(NBUF):
            j = i * NBUF + b
            process(j, b)
            drain(j - (NBUF - LOOK), (b + LOOK) % NBUF)
            issue(j + LOOK, (b + LOOK) % NBUF)
        return _

    lax.fori_loop(0, -(-TOTJ // NBUF), step, None)
    for t in range(NBUF - LOOK):
        jj = -(-TOTJ // NBUF) * NBUF - (NBUF - LOOK) + t
        drain(jj, jj % NBUF)

    pltpu.sync_copy(cnt_v, cnt_sh.at[iota_v], add=True)
    plsc.subcore_barrier()

    pltpu.sync_copy(sums_sh.at[pl.ds(g0, GPT)], sums4_v)
    pltpu.sync_copy(cnt_sh, cnt64_v)
    for g in range(GPT):
        cvec = plsc.load_gather(cnt64_v, [jnp.full((16,), g0 + g, jnp.int32)])
        for j in range(CH // 16):
            out_v[g, pl.ds(j * 16, 16)] = sums4_v[g, pl.ds(j * 16, 16)] / cvec
    pltpu.sync_copy(out_v, out_hbm.at[pl.ds(g0, GPT), pl.ds(c0, CH)])


@jax.jit
def _pooling(x, batch):
    mesh = plsc.VectorSubcoreMesh(core_axis_name="c", subcore_axis_name="s")
    f = pl.kernel(
        _body,
        out_type=jax.ShapeDtypeStruct((G, C), jnp.float32),
        mesh=mesh,
        compiler_params=pltpu.CompilerParams(use_tc_tiling_on_sc=False,
                                             needs_layout_passes=False,
                                             disable_bounds_checks=True,
                                             disable_semaphore_checks=True,
                                             skip_device_barrier=True),
        scratch_types=[
            [pltpu.VMEM((SUBC, ROW), jnp.int32) for _ in range(NBUF)],
            [pltpu.VMEM((CHUNK, CH), jnp.float32) for _ in range(NBUF)],
            pltpu.VMEM((G,), jnp.float32),
            pltpu.VMEM((G,), jnp.int32),
            pltpu.VMEM((GPT, CH), jnp.float32),
            pltpu.VMEM((GPT, CH), jnp.float32),
            pltpu.VMEM((G,), jnp.float32),
            pltpu.VMEM((GPT, CH), jnp.float32),
            pltpu.VMEM_SHARED((G, CH), jnp.float32),
            pltpu.VMEM_SHARED((G,), jnp.float32),
            [pltpu.SemaphoreType.DMA for _ in range(NBUF)],
            [pltpu.SemaphoreType.DMA for _ in range(NBUF)],
        ],
    )
    return f(x, batch)


def kernel(x, batch):
    return _pooling(x, batch.astype(jnp.int32).reshape(N // ROW, ROW))
